# baseline (speedup 1.0000x reference)
import functools

import jax
import jax.numpy as jnp
from jax import lax
from jax.experimental import pallas as pl
from jax.experimental.pallas import tpu as pltpu
from jax.experimental.pallas import tpu_sc as plsc

D = 768
DF = 4 * D
VOCAB = 21128
NR = 4
NS = 2
NE = NR + NS
S = 2048
NCHUNK = 1024
NN = DF // NCHUNK
VB = 2688
NV = (VOCAB + VB - 1) // VB
MB = 512
NM = S // MB


def _gelu(x):
    return x * 0.5 * (1.0 + lax.erf(x * 0.7071067811865476))


def _mm(x, w_ref, lowp):
    if lowp:
        return jnp.dot(x.astype(jnp.bfloat16), w_ref[...].astype(jnp.bfloat16),
                       preferred_element_type=jnp.float32)
    return jnp.dot(x, w_ref[...], preferred_element_type=jnp.float32)


def _ln(x, g, b, eps):
    m = jnp.mean(x, axis=-1, keepdims=True)
    v = jnp.mean(jnp.square(x - m), axis=-1, keepdims=True)
    return (x - m) / jnp.sqrt(v + eps) * g + b


def _ln_na(x, eps):
    m = jnp.mean(x, axis=-1, keepdims=True)
    v = jnp.mean(jnp.square(x - m), axis=-1, keepdims=True)
    return (x - m) / jnp.sqrt(v + eps)


def _emb_gather(table, idx):
    info = plsc.get_sparse_core_info()
    nw = info.num_cores * info.num_subcores
    b_per_w = S // nw
    mesh = plsc.VectorSubcoreMesh(core_axis_name="c", subcore_axis_name="s")

    @functools.partial(
        pl.kernel, mesh=mesh,
        out_type=jax.ShapeDtypeStruct((S, D), jnp.float32),
        scratch_types=[
            pltpu.VMEM((b_per_w,), jnp.int32),
            pltpu.VMEM((b_per_w, D), jnp.float32),
            pltpu.SemaphoreType.DMA,
        ],
    )
    def k(table_hbm, idx_hbm, out_hbm, idx_v, rows_v, sem):
        wid = lax.axis_index("s") * info.num_cores + lax.axis_index("c")
        base = wid * b_per_w
        pltpu.sync_copy(idx_hbm.at[pl.ds(base, b_per_w)], idx_v)
        pltpu.async_copy(table_hbm.at[idx_v], rows_v, sem).wait()
        pltpu.sync_copy(rows_v, out_hbm.at[pl.ds(base, b_per_w)])

    return k(table, idx)


def _pre_body(rows_ref, pos_ref, g_ref, b_ref, o_ref):
    x = jnp.clip(rows_ref[...], -100.0, 100.0) + jnp.clip(pos_ref[...], -100.0, 100.0)
    o_ref[...] = _ln(x, g_ref[...], b_ref[...], 1e-5)


def _pre(rows, pos, g, b):
    return pl.pallas_call(
        _pre_body,
        out_shape=jax.ShapeDtypeStruct((S, D), jnp.float32),
    )(rows, pos, g.reshape(1, D), b.reshape(1, D))


def _router_body(h_ref, w1_ref, b1_ref, w2_ref, b2_ref, wall_ref):
    r = _gelu(jnp.dot(h_ref[...], w1_ref[...],
                      preferred_element_type=jnp.float32) + b1_ref[...])
    logits = jnp.dot(r, w2_ref[...], preferred_element_type=jnp.float32) + b2_ref[...]
    io = lax.broadcasted_iota(jnp.int32, (S, NR), 1)
    v1 = jnp.max(logits, axis=1, keepdims=True)
    i1 = jnp.min(jnp.where(logits == v1, io, NR), axis=1, keepdims=True)
    neg = jnp.where(io == i1, -1e30, logits)
    v2 = jnp.max(neg, axis=1, keepdims=True)
    i2 = jnp.min(jnp.where(neg == v2, io, NR), axis=1, keepdims=True)
    e2 = jnp.exp(v2 - v1)
    w1 = 1.0 / (1.0 + e2)
    w2 = e2 / (1.0 + e2)
    io6 = lax.broadcasted_iota(jnp.int32, (S, NE), 1)
    wall = (w1 * (io6 == i1) + w2 * (io6 == i2)
            + jnp.where(io6 >= NR, 1.0 / NS, 0.0))
    wall_ref[...] = wall


def _router(h, w1, b1, w2, b2):
    return pl.pallas_call(
        _router_body,
        out_shape=jax.ShapeDtypeStruct((S, NE), jnp.float32),
    )(h, w1, b1, w2, b2)


def _expert_first_body(h_ref, wall_ref, w1_ref, b1_ref, w2_ref, b2_ref,
                       out_ref, *, j, lowp):
    n = pl.program_id(0)
    t = _gelu(_mm(h_ref[...], w1_ref, lowp) + b1_ref[...])
    part = _mm(t, w2_ref, lowp)
    io6 = lax.broadcasted_iota(jnp.int32, (S, NE), 1)
    w = jnp.sum(wall_ref[...] * (io6 == j), axis=1, keepdims=True)

    @pl.when(n == 0)
    def _():
        out_ref[...] = w * (part + b2_ref[...])

    @pl.when(n != 0)
    def _():
        out_ref[...] += w * part


def _expert_acc_body(h_ref, wall_ref, w1_ref, b1_ref, w2_ref, b2_ref,
                     acc_ref, out_ref, *, j, lowp):
    n = pl.program_id(0)
    t = _gelu(_mm(h_ref[...], w1_ref, lowp) + b1_ref[...])
    part = _mm(t, w2_ref, lowp)
    io6 = lax.broadcasted_iota(jnp.int32, (S, NE), 1)
    w = jnp.sum(wall_ref[...] * (io6 == j), axis=1, keepdims=True)

    @pl.when(n == 0)
    def _():
        out_ref[...] = acc_ref[...] + w * (part + b2_ref[...])

    @pl.when(n != 0)
    def _():
        out_ref[...] += w * part


_EXPERT_SPECS = [
    pl.BlockSpec((S, D), lambda n: (0, 0)),
    pl.BlockSpec((S, NE), lambda n: (0, 0)),
    pl.BlockSpec((D, NCHUNK), lambda n: (0, n)),
    pl.BlockSpec((1, NCHUNK), lambda n: (0, n)),
    pl.BlockSpec((NCHUNK, D), lambda n: (n, 0)),
    pl.BlockSpec((1, D), lambda n: (0, 0)),
]


def _expert(h, wall, ex, j, acc, lowp):
    w1 = ex["l1"]["w"]
    b1 = ex["l1"]["b"].reshape(1, DF)
    w2 = ex["l2"]["w"]
    b2 = ex["l2"]["b"].reshape(1, D)
    if acc is None:
        return pl.pallas_call(
            functools.partial(_expert_first_body, j=j, lowp=lowp),
            grid=(NN,),
            in_specs=_EXPERT_SPECS,
            out_specs=pl.BlockSpec((S, D), lambda n: (0, 0)),
            out_shape=jax.ShapeDtypeStruct((S, D), jnp.float32),
        )(h, wall, w1, b1, w2, b2)
    return pl.pallas_call(
        functools.partial(_expert_acc_body, j=j, lowp=lowp),
        grid=(NN,),
        in_specs=_EXPERT_SPECS + [pl.BlockSpec((S, D), lambda n: (0, 0))],
        out_specs=pl.BlockSpec((S, D), lambda n: (0, 0)),
        out_shape=jax.ShapeDtypeStruct((S, D), jnp.float32),
        input_output_aliases={6: 0},
    )(h, wall, w1, b1, w2, b2, acc)


def _combine_body(h_ref, f_ref, w_ref, b_ref, g1_ref, bb1_ref, o_ref, *, lowp):
    out = _mm(f_ref[...], w_ref, lowp) + b_ref[...]
    out = out * 0.5 + h_ref[...] * 0.5
    a = _ln_na(out, 1e-6)
    o_ref[...] = _ln(h_ref[...] + a, g1_ref[...], bb1_ref[...], 1e-5)


def _combine(h, final, w, b, g1, b1, lowp):
    return pl.pallas_call(
        functools.partial(_combine_body, lowp=lowp),
        out_shape=jax.ShapeDtypeStruct((S, D), jnp.float32),
    )(h, final, w, b.reshape(1, D), g1.reshape(1, D), b1.reshape(1, D))


def _mlp_body(h_ref, wi_ref, bi_ref, wo_ref, bo_ref, g_ref, b_ref,
              o_ref, acc_ref, *, lowp):
    n = pl.program_id(0)

    @pl.when(n == 0)
    def _():
        acc_ref[...] = jnp.zeros_like(acc_ref)

    t = _gelu(_mm(h_ref[...], wi_ref, lowp) + bi_ref[...])
    acc_ref[...] += _mm(t, wo_ref, lowp)

    @pl.when(n == NN - 1)
    def _():
        o = acc_ref[...] + bo_ref[...]
        o_ref[...] = _ln(h_ref[...] + o, g_ref[...], b_ref[...], 1e-5)


def _mlp(h, wi, bi, wo, bo, g, b, lowp):
    return pl.pallas_call(
        functools.partial(_mlp_body, lowp=lowp),
        grid=(NN,),
        in_specs=[
            pl.BlockSpec((S, D), lambda n: (0, 0)),
            pl.BlockSpec((D, NCHUNK), lambda n: (0, n)),
            pl.BlockSpec((1, NCHUNK), lambda n: (0, n)),
            pl.BlockSpec((NCHUNK, D), lambda n: (n, 0)),
            pl.BlockSpec((1, D), lambda n: (0, 0)),
            pl.BlockSpec((1, D), lambda n: (0, 0)),
            pl.BlockSpec((1, D), lambda n: (0, 0)),
        ],
        out_specs=pl.BlockSpec((S, D), lambda n: (0, 0)),
        out_shape=jax.ShapeDtypeStruct((S, D), jnp.float32),
        scratch_shapes=[pltpu.VMEM((S, D), jnp.float32)],
    )(h, wi, bi.reshape(1, DF), wo, bo.reshape(1, D), g.reshape(1, D),
      b.reshape(1, D))


def _lm_body(h_ref, w_ref, b_ref, o_ref):
    o_ref[...] = _mm(h_ref[...], w_ref, True) + b_ref[...]


def _lm_head(h, w, b):
    return pl.pallas_call(
        _lm_body,
        grid=(NV, NM),
        in_specs=[
            pl.BlockSpec((MB, D), lambda v, m: (m, 0)),
            pl.BlockSpec((D, VB), lambda v, m: (0, v)),
            pl.BlockSpec((1, VB), lambda v, m: (0, v)),
        ],
        out_specs=pl.BlockSpec((MB, VB), lambda v, m: (m, v)),
        out_shape=jax.ShapeDtypeStruct((S, VOCAB), jnp.float32),
    )(h, w, b.reshape(1, VOCAB))


def kernel(input_ids, params):
    p = params
    ids = input_ids.reshape(-1).astype(jnp.int32)
    rows = jnp.take(p["emb"], ids, axis=0)
    h = _pre(rows, p["pos"][:S], p["ln_g"], p["ln_b"])
    for li, lp in enumerate(p["layers"]):
        lowp = li > 0
        wall = _router(h, lp["router1"]["w"], lp["router1"]["b"].reshape(1, D),
                       lp["router2"]["w"], lp["router2"]["b"].reshape(1, NR))
        experts = list(lp["routed"]) + list(lp["shared"])
        final = None
        for j, ex in enumerate(experts):
            final = _expert(h, wall, ex, j, final, lowp)
        h = _combine(h, final, lp["attn_out"]["w"], lp["attn_out"]["b"],
                     lp["ln1_g"], lp["ln1_b"], lowp)
        h = _mlp(h, lp["inter"]["w"], lp["inter"]["b"], lp["out"]["w"],
                 lp["out"]["b"], lp["ln2_g"], lp["ln2_b"], lowp)
    logits = _lm_head(h, p["lm_head"]["w"], p["lm_head"]["b"])
    return logits.reshape(1, S, VOCAB)


# speedup vs baseline: 1.2859x; 1.2859x over previous
import functools

import jax
import jax.numpy as jnp
from jax import lax
from jax.experimental import pallas as pl
from jax.experimental.pallas import tpu as pltpu
from jax.experimental.pallas import tpu_sc as plsc

D = 768
DF = 4 * D
VOCAB = 21128
NR = 4
NS = 2
NE = NR + NS
S = 2048
NCHUNK = 1024
NN = DF // NCHUNK
VB = 2688
NV = (VOCAB + VB - 1) // VB
MB = 512
NM = S // MB


def _gelu(x):
    return x * 0.5 * (1.0 + lax.erf(x * 0.7071067811865476))


def _mmv(x, w, lowp):
    if lowp:
        return jnp.dot(x.astype(jnp.bfloat16), w.astype(jnp.bfloat16),
                       preferred_element_type=jnp.float32)
    return jnp.dot(x, w, preferred_element_type=jnp.float32)


def _ln(x, g, b, eps):
    m = jnp.mean(x, axis=-1, keepdims=True)
    v = jnp.mean(jnp.square(x - m), axis=-1, keepdims=True)
    return (x - m) / jnp.sqrt(v + eps) * g + b


def _ln_na(x, eps):
    m = jnp.mean(x, axis=-1, keepdims=True)
    v = jnp.mean(jnp.square(x - m), axis=-1, keepdims=True)
    return (x - m) / jnp.sqrt(v + eps)


def _emb_gather(table, idx):
    info = plsc.get_sparse_core_info()
    nw = info.num_cores * info.num_subcores
    b_per_w = S // nw
    mesh = plsc.VectorSubcoreMesh(core_axis_name="c", subcore_axis_name="s")

    @functools.partial(
        pl.kernel, mesh=mesh,
        out_type=jax.ShapeDtypeStruct((S, D), jnp.float32),
        scratch_types=[
            pltpu.VMEM((b_per_w,), jnp.int32),
            pltpu.VMEM((b_per_w, D), jnp.float32),
            pltpu.SemaphoreType.DMA,
        ],
    )
    def k(table_hbm, idx_hbm, out_hbm, idx_v, rows_v, sem):
        wid = lax.axis_index("s") * info.num_cores + lax.axis_index("c")
        base = wid * b_per_w
        pltpu.sync_copy(idx_hbm.at[pl.ds(base, b_per_w)], idx_v)
        pltpu.async_copy(table_hbm.at[idx_v], rows_v, sem).wait()
        pltpu.sync_copy(rows_v, out_hbm.at[pl.ds(base, b_per_w)])

    return k(table, idx)


def _pre_body(rows_ref, pos_ref, g_ref, b_ref, o_ref):
    x = jnp.clip(rows_ref[...], -100.0, 100.0) + jnp.clip(pos_ref[...], -100.0, 100.0)
    o_ref[...] = _ln(x, g_ref[...], b_ref[...], 1e-5)


def _pre(rows, pos, g, b):
    return pl.pallas_call(
        _pre_body,
        out_shape=jax.ShapeDtypeStruct((S, D), jnp.float32),
    )(rows, pos, g.reshape(1, D), b.reshape(1, D))


NSTEP = (NE + 1) * NN


def _layer_body(h_ref, r1w_ref, r1b_ref, r2w_ref, r2b_ref,
                b1s_ref, b2s_ref, attnw_ref, attnb_ref, g1_ref, bb1_ref,
                bi_ref, bo_ref, g2_ref, bb2_ref,
                w10, w11, w12, w13, w14, w15,
                w20, w21, w22, w23, w24, w25,
                wi_any, wo_any,
                out_ref,
                w1buf, w2buf, acc_ref, h1_ref, wall_ref, sem1, sem2, *, lowp):
    w1s = [w10, w11, w12, w13, w14, w15, wi_any]
    w2s = [w20, w21, w22, w23, w24, w25, wo_any]

    def src1(k):
        j, n = k // NN, k % NN
        return w1s[j].at[:, pl.ds(n * NCHUNK, NCHUNK)]

    def src2(k):
        j, n = k // NN, k % NN
        return w2s[j].at[pl.ds(n * NCHUNK, NCHUNK), :]

    def start(k):
        b = k % 2
        pltpu.make_async_copy(src1(k), w1buf.at[b], sem1.at[b]).start()
        pltpu.make_async_copy(src2(k), w2buf.at[b], sem2.at[b]).start()

    def wait(k):
        b = k % 2
        pltpu.make_async_copy(src1(k), w1buf.at[b], sem1.at[b]).wait()
        pltpu.make_async_copy(src2(k), w2buf.at[b], sem2.at[b]).wait()

    start(0)
    start(1)

    HB = S // 2
    io = lax.broadcasted_iota(jnp.int32, (HB, NR), 1)
    io6 = lax.broadcasted_iota(jnp.int32, (HB, NE), 1)

    for i in range(2):
        sl = pl.ds(i * HB, HB)
        xh = h_ref[sl, :]
        r = _gelu(jnp.dot(xh, r1w_ref[...], preferred_element_type=jnp.float32)
                  + r1b_ref[...])
        logits = (jnp.dot(r, r2w_ref[...], preferred_element_type=jnp.float32)
                  + r2b_ref[...])
        v1 = jnp.max(logits, axis=1, keepdims=True)
        i1 = jnp.min(jnp.where(logits == v1, io, NR), axis=1, keepdims=True)
        neg = jnp.where(io == i1, -1e30, logits)
        v2 = jnp.max(neg, axis=1, keepdims=True)
        i2 = jnp.min(jnp.where(neg == v2, io, NR), axis=1, keepdims=True)
        e2 = jnp.exp(v2 - v1)
        wall_ref[sl, :] = ((1.0 / (1.0 + e2)) * (io6 == i1)
                           + (e2 / (1.0 + e2)) * (io6 == i2)
                           + jnp.where(io6 >= NR, 1.0 / NS, 0.0))

    for k in range(NE * NN):
        j, n = k // NN, k % NN
        b = k % 2
        wait(k)
        for i in range(2):
            sl = pl.ds(i * HB, HB)
            t = _gelu(_mmv(h_ref[sl, :], w1buf[b], lowp)
                      + b1s_ref[j, :, n * NCHUNK:(n + 1) * NCHUNK])
            part = _mmv(t, w2buf[b], lowp)
            w = jnp.sum(wall_ref[sl, :] * (io6 == j), axis=1, keepdims=True)
            if k == 0:
                acc_ref[sl, :] = w * (part + b2s_ref[j])
            elif n == 0:
                acc_ref[sl, :] += w * (part + b2s_ref[j])
            else:
                acc_ref[sl, :] += w * part
        if k + 2 < NSTEP:
            start(k + 2)

    for i in range(2):
        sl = pl.ds(i * HB, HB)
        out = _mmv(acc_ref[sl, :], attnw_ref[...], lowp) + attnb_ref[...]
        out = out * 0.5 + h_ref[sl, :] * 0.5
        a = _ln_na(out, 1e-6)
        h1_ref[sl, :] = _ln(h_ref[sl, :] + a, g1_ref[...], bb1_ref[...], 1e-5)

    for n in range(NN):
        k = NE * NN + n
        b = k % 2
        wait(k)
        for i in range(2):
            sl = pl.ds(i * HB, HB)
            t = _gelu(_mmv(h1_ref[sl, :], w1buf[b], lowp)
                      + bi_ref[:, n * NCHUNK:(n + 1) * NCHUNK])
            part = _mmv(t, w2buf[b], lowp)
            if n == 0:
                acc_ref[sl, :] = part
            else:
                acc_ref[sl, :] += part
        if k + 2 < NSTEP:
            start(k + 2)

    for i in range(2):
        sl = pl.ds(i * HB, HB)
        o = acc_ref[sl, :] + bo_ref[...]
        out_ref[sl, :] = _ln(h1_ref[sl, :] + o, g2_ref[...], bb2_ref[...], 1e-5)


def _layer(h, lp, lowp):
    experts = list(lp["routed"]) + list(lp["shared"])
    b1s = jnp.stack([e["l1"]["b"].reshape(1, DF) for e in experts])
    b2s = jnp.stack([e["l2"]["b"].reshape(1, D) for e in experts])
    vmem = pl.BlockSpec(memory_space=pl.ANY)
    any_specs = [vmem] * 14
    n_vmem = 15
    in_specs = ([pl.BlockSpec(memory_space=pltpu.MemorySpace.VMEM)] * n_vmem + any_specs)
    return pl.pallas_call(
        functools.partial(_layer_body, lowp=lowp),
        in_specs=in_specs,
        out_specs=pl.BlockSpec(memory_space=pltpu.MemorySpace.VMEM),
        out_shape=jax.ShapeDtypeStruct((S, D), jnp.float32),
        scratch_shapes=[
            pltpu.VMEM((2, D, NCHUNK), jnp.float32),
            pltpu.VMEM((2, NCHUNK, D), jnp.float32),
            pltpu.VMEM((S, D), jnp.float32),
            pltpu.VMEM((S, D), jnp.float32),
            pltpu.VMEM((S, NE), jnp.float32),
            pltpu.SemaphoreType.DMA((2,)),
            pltpu.SemaphoreType.DMA((2,)),
        ],
    )(h, lp["router1"]["w"], lp["router1"]["b"].reshape(1, D),
      lp["router2"]["w"], lp["router2"]["b"].reshape(1, NR),
      b1s, b2s,
      lp["attn_out"]["w"], lp["attn_out"]["b"].reshape(1, D),
      lp["ln1_g"].reshape(1, D), lp["ln1_b"].reshape(1, D),
      lp["inter"]["b"].reshape(1, DF), lp["out"]["b"].reshape(1, D),
      lp["ln2_g"].reshape(1, D), lp["ln2_b"].reshape(1, D),
      *[e["l1"]["w"] for e in experts],
      *[e["l2"]["w"] for e in experts],
      lp["inter"]["w"], lp["out"]["w"])


def _lm_body(h_ref, w_ref, b_ref, o_ref):
    o_ref[...] = _mmv(h_ref[...], w_ref[...], True) + b_ref[...]


def _lm_head(h, w, b):
    return pl.pallas_call(
        _lm_body,
        grid=(NV, NM),
        in_specs=[
            pl.BlockSpec((MB, D), lambda v, m: (m, 0)),
            pl.BlockSpec((D, VB), lambda v, m: (0, v)),
            pl.BlockSpec((1, VB), lambda v, m: (0, v)),
        ],
        out_specs=pl.BlockSpec((MB, VB), lambda v, m: (m, v)),
        out_shape=jax.ShapeDtypeStruct((S, VOCAB), jnp.float32),
    )(h, w, b.reshape(1, VOCAB))


def kernel(input_ids, params):
    p = params
    ids = input_ids.reshape(-1).astype(jnp.int32)
    rows = _emb_gather(p["emb"], ids)
    h = _pre(rows, p["pos"][:S], p["ln_g"], p["ln_b"])
    for li, lp in enumerate(p["layers"]):
        h = _layer(h, lp, li > 0)
    logits = _lm_head(h, p["lm_head"]["w"], p["lm_head"]["b"])
    return logits.reshape(1, S, VOCAB)
